# Initial kernel scaffold; baseline (speedup 1.0000x reference)
import jax
import jax.numpy as jnp
from jax.experimental import pallas as pl


def kernel(x, edge_index, edge_attr, W, b, W2, b2):
    raise NotImplementedError("write your pallas kernel here")


# speedup vs baseline: 25.7340x; 25.7340x over previous
import functools

import jax
import jax.numpy as jnp
from jax import lax
from jax.experimental import pallas as pl
from jax.experimental.pallas import tpu as pltpu
from jax.experimental.pallas import tpu_sc as plsc

_N = 10000
_E = 320000
_D = 128

_NC = 2
_NS = 16
_NW = _NC * _NS
_EPW = _E // _NW
_CHUNK = 80
_NCHUNK = _EPW // _CHUNK
_RPT = 1000
_NEXP = _N // _RPT

_mesh = plsc.VectorSubcoreMesh(core_axis_name="c", subcore_axis_name="s")


@functools.partial(
    pl.kernel,
    out_type=jax.ShapeDtypeStruct((_NC, _N, 16), jnp.float32),
    mesh=_mesh,
    scratch_types=[
        pltpu.VMEM((_NCHUNK, _CHUNK), jnp.int32),
        pltpu.VMEM((_CHUNK, 16), jnp.float32),
        pltpu.VMEM_SHARED((_N, 16), jnp.float32),
    ],
)
def _deg_kernel(col_hbm, ones_hbm, zeros_hbm, out_hbm, col_v, ones_v, deg_sh):
    cid = lax.axis_index("c")
    sid = lax.axis_index("s")
    wid = cid * _NS + sid
    pltpu.sync_copy(col_hbm.at[wid], col_v)
    pltpu.sync_copy(ones_hbm, ones_v)

    @pl.when(sid < _NEXP)
    def _zero():
        pltpu.sync_copy(zeros_hbm.at[pl.ds(sid * _RPT, _RPT)],
                        deg_sh.at[pl.ds(sid * _RPT, _RPT)])

    plsc.subcore_barrier()

    def body(j, carry):
        pltpu.sync_copy(ones_v, deg_sh.at[col_v.at[j]], add=True)
        return carry

    lax.fori_loop(0, _NCHUNK, body, 0)
    plsc.subcore_barrier()

    @pl.when(sid < _NEXP)
    def _export():
        pltpu.sync_copy(deg_sh.at[pl.ds(sid * _RPT, _RPT)],
                        out_hbm.at[cid, pl.ds(sid * _RPT, _RPT)])


@functools.partial(
    pl.kernel,
    out_type=jax.ShapeDtypeStruct((_NC, _N, _D), jnp.float32),
    mesh=_mesh,
    scratch_types=[
        pltpu.VMEM((_NCHUNK, _CHUNK), jnp.int32),
        pltpu.VMEM((_NCHUNK, _CHUNK), jnp.int32),
        pltpu.VMEM((_CHUNK, _D), jnp.float32),
        pltpu.VMEM_SHARED((_N, _D), jnp.float32),
        pltpu.SemaphoreType.DMA,
    ],
)
def _msg_kernel(g_hbm, row_hbm, col_hbm, zeros_hbm, out_hbm,
                row_v, col_v, buf, acc_sh, sem):
    cid = lax.axis_index("c")
    sid = lax.axis_index("s")
    wid = cid * _NS + sid
    pltpu.sync_copy(row_hbm.at[wid], row_v)
    pltpu.sync_copy(col_hbm.at[wid], col_v)

    @pl.when(sid < _NEXP)
    def _zero():
        pltpu.sync_copy(zeros_hbm.at[pl.ds(sid * _RPT, _RPT)],
                        acc_sh.at[pl.ds(sid * _RPT, _RPT)])

    plsc.subcore_barrier()

    def body(j, carry):
        pltpu.async_copy(g_hbm.at[row_v.at[j]], buf, sem).wait()
        pltpu.sync_copy(buf, acc_sh.at[col_v.at[j]], add=True)
        return carry

    lax.fori_loop(0, _NCHUNK, body, 0)
    plsc.subcore_barrier()

    @pl.when(sid < _NEXP)
    def _export():
        pltpu.sync_copy(acc_sh.at[pl.ds(sid * _RPT, _RPT)],
                        out_hbm.at[cid, pl.ds(sid * _RPT, _RPT)])


_ROWS = 2000


def _mm_body(x_ref, w_ref, h_ref):
    h_ref[...] = jnp.dot(x_ref[...], w_ref[...],
                         preferred_element_type=jnp.float32)


def _norm_body(parts_ref, h_ref, g_ref):
    deg = 1.0 + jnp.sum(parts_ref[...], axis=(0, 2)) * (1.0 / 16.0)
    p = lax.rsqrt(deg)
    g_ref[...] = p[:, None] * h_ref[...]


def _head_body(parts_ref, acc_ref, g_ref, b_ref, w2_ref, b2_ref, y_ref):
    deg = 1.0 + jnp.sum(parts_ref[...], axis=(0, 2)) * (1.0 / 16.0)
    p = lax.rsqrt(deg)
    t = p[:, None] * (acc_ref[0] + acc_ref[1] + g_ref[...]) + b_ref[...]
    t = jnp.maximum(t, 0.0)
    y_ref[...] = jnp.dot(t, w2_ref[...],
                         preferred_element_type=jnp.float32) + b2_ref[...]


def kernel(x, edge_index, edge_attr, W, b, W2, b2):
    del edge_attr
    row = edge_index[0].astype(jnp.int32).reshape(_NW, _NCHUNK, _CHUNK)
    col = edge_index[1].astype(jnp.int32).reshape(_NW, _NCHUNK, _CHUNK)
    ones16 = jnp.ones((_CHUNK, 16), jnp.float32)
    zeros16 = jnp.zeros((_N, 16), jnp.float32)
    zerosd = jnp.zeros((_N, _D), jnp.float32)

    deg_parts = _deg_kernel(col, ones16, zeros16)

    h = pl.pallas_call(
        _mm_body,
        grid=(_N // _ROWS,),
        in_specs=[
            pl.BlockSpec((_ROWS, _D), lambda i: (i, 0)),
            pl.BlockSpec((_D, _D), lambda i: (0, 0)),
        ],
        out_specs=pl.BlockSpec((_ROWS, _D), lambda i: (i, 0)),
        out_shape=jax.ShapeDtypeStruct((_N, _D), jnp.float32),
    )(x, W)

    g = pl.pallas_call(
        _norm_body,
        grid=(_N // _ROWS,),
        in_specs=[
            pl.BlockSpec((_NC, _ROWS, 16), lambda i: (0, i, 0)),
            pl.BlockSpec((_ROWS, _D), lambda i: (i, 0)),
        ],
        out_specs=pl.BlockSpec((_ROWS, _D), lambda i: (i, 0)),
        out_shape=jax.ShapeDtypeStruct((_N, _D), jnp.float32),
    )(deg_parts, h)

    acc = _msg_kernel(g, row, col, zerosd)

    y = pl.pallas_call(
        _head_body,
        grid=(_N // _ROWS,),
        in_specs=[
            pl.BlockSpec((_NC, _ROWS, 16), lambda i: (0, i, 0)),
            pl.BlockSpec((_NC, _ROWS, _D), lambda i: (0, i, 0)),
            pl.BlockSpec((_ROWS, _D), lambda i: (i, 0)),
            pl.BlockSpec((_D,), lambda i: (0,)),
            pl.BlockSpec((_D, 1), lambda i: (0, 0)),
            pl.BlockSpec((1,), lambda i: (0,)),
        ],
        out_specs=pl.BlockSpec((_ROWS, 1), lambda i: (i, 0)),
        out_shape=jax.ShapeDtypeStruct((_N, 1), jnp.float32),
    )(deg_parts, acc, g, b, W2, b2)
    return y
